# baseline (speedup 1.0000x reference)
import functools

import jax
import jax.numpy as jnp
from jax import lax
from jax.experimental import pallas as pl
from jax.experimental.pallas import tpu as pltpu
from jax.experimental.pallas import tpu_sc as plsc

VOCAB = 100000
WORD_DIM = 300
EMBED_DIM = 128
TOK_B, TOK_L = 4096, 50
N_TOK = TOK_B * TOK_L

_PROJ_TILE = 2560


def _proj_body(wvt_ref, wt_ref, out_ref):
    out_ref[...] = lax.dot_general(
        wvt_ref[...],
        wt_ref[...],
        dimension_numbers=(((0,), (0,)), ((), ())),
        preferred_element_type=jnp.float32,
    )


def _project_table(wv_t, w_t):
    grid = pl.cdiv(VOCAB, _PROJ_TILE)
    return pl.pallas_call(
        _proj_body,
        grid=(grid,),
        in_specs=[
            pl.BlockSpec((WORD_DIM, _PROJ_TILE), lambda i: (0, i)),
            pl.BlockSpec((WORD_DIM, EMBED_DIM), lambda i: (0, 0)),
        ],
        out_specs=pl.BlockSpec((_PROJ_TILE, EMBED_DIM), lambda i: (i, 0)),
        out_shape=jax.ShapeDtypeStruct((VOCAB, EMBED_DIM), jnp.float32),
    )(wv_t, w_t)


def _make_gather():
    info = plsc.get_sparse_core_info()
    nc, ns = info.num_cores, info.num_subcores
    nw = nc * ns
    b_per_w = N_TOK // nw
    chunk = 400
    n_chunks = b_per_w // chunk
    mesh = plsc.VectorSubcoreMesh(core_axis_name="c", subcore_axis_name="s")

    @functools.partial(
        pl.kernel,
        mesh=mesh,
        out_type=jax.ShapeDtypeStruct((N_TOK, EMBED_DIM), jnp.float32),
        scratch_types=[
            pltpu.VMEM((b_per_w,), jnp.int32),
            pltpu.VMEM((2, chunk, EMBED_DIM), jnp.float32),
            pltpu.SemaphoreType.DMA,
            pltpu.SemaphoreType.DMA,
        ],
    )
    def gather_kernel(table_hbm, idx_hbm, out_hbm, idx_v, rows_v, gsem, ssem):
        wid = lax.axis_index("s") * nc + lax.axis_index("c")
        base = wid * b_per_w
        pltpu.sync_copy(idx_hbm.at[pl.ds(base, b_per_w)], idx_v)

        def gather_copy(c, buf):
            return pltpu.make_async_copy(
                table_hbm.at[idx_v.at[pl.ds(c * chunk, chunk)]],
                rows_v.at[buf],
                gsem,
            )

        def store_copy(c, buf):
            return pltpu.make_async_copy(
                rows_v.at[buf],
                out_hbm.at[pl.ds(base + c * chunk, chunk)],
                ssem,
            )

        gather_copy(0, 0).start()

        def body(c, carry):
            buf = lax.rem(c, 2)
            nxt = lax.rem(c + 1, 2)

            @pl.when(c > 0)
            def _():
                store_copy(c - 1, nxt).wait()

            @pl.when(c < n_chunks - 1)
            def _():
                gather_copy(c + 1, nxt).start()

            gather_copy(c, buf).wait()
            store_copy(c, buf).start()
            return carry

        lax.fori_loop(0, n_chunks, body, 0)
        store_copy(n_chunks - 1, lax.rem(n_chunks - 1, 2)).wait()

    return gather_kernel


def kernel(x, word_vectors, W):
    projected = _project_table(word_vectors.T, W.T)
    idx = x.T.reshape(N_TOK).astype(jnp.int32)
    out = _make_gather()(projected, idx)
    return out.reshape(TOK_L, TOK_B, EMBED_DIM).transpose(1, 0, 2)


# speedup vs baseline: 1.0880x; 1.0880x over previous
import functools

import jax
import jax.numpy as jnp
from jax import lax
from jax.experimental import pallas as pl
from jax.experimental.pallas import tpu as pltpu
from jax.experimental.pallas import tpu_sc as plsc

VOCAB = 100000
WORD_DIM = 300
EMBED_DIM = 128
TOK_B, TOK_L = 4096, 50
N_TOK = TOK_B * TOK_L

_PROJ_TILE = 12800


def _proj_body(wvt_ref, wt_ref, out_ref):
    out_ref[...] = lax.dot_general(
        wvt_ref[...],
        wt_ref[...],
        dimension_numbers=(((0,), (0,)), ((), ())),
        preferred_element_type=jnp.float32,
    )


def _project_table(wv_t, w_t):
    grid = pl.cdiv(VOCAB, _PROJ_TILE)
    return pl.pallas_call(
        _proj_body,
        grid=(grid,),
        in_specs=[
            pl.BlockSpec((WORD_DIM, _PROJ_TILE), lambda i: (0, i)),
            pl.BlockSpec((WORD_DIM, EMBED_DIM), lambda i: (0, 0)),
        ],
        out_specs=pl.BlockSpec((_PROJ_TILE, EMBED_DIM), lambda i: (i, 0)),
        out_shape=jax.ShapeDtypeStruct((VOCAB, EMBED_DIM), jnp.float32),
    )(wv_t, w_t)


def _make_gather():
    info = plsc.get_sparse_core_info()
    nc, ns = info.num_cores, info.num_subcores
    nw = nc * ns
    b_per_w = N_TOK // nw
    chunk = 400
    n_chunks = b_per_w // chunk
    mesh = plsc.VectorSubcoreMesh(core_axis_name="c", subcore_axis_name="s")

    @functools.partial(
        pl.kernel,
        mesh=mesh,
        out_type=jax.ShapeDtypeStruct((N_TOK, EMBED_DIM), jnp.float32),
        scratch_types=[
            pltpu.VMEM((b_per_w,), jnp.int32),
            pltpu.VMEM((2, chunk, EMBED_DIM), jnp.float32),
            pltpu.SemaphoreType.DMA,
            pltpu.SemaphoreType.DMA,
        ],
    )
    def gather_kernel(table_hbm, idx_hbm, out_hbm, idx_v, rows_v, gsem, ssem):
        wid = lax.axis_index("s") * nc + lax.axis_index("c")
        base = wid * b_per_w
        pltpu.sync_copy(idx_hbm.at[pl.ds(base, b_per_w)], idx_v)

        def gather_copy(c, buf):
            return pltpu.make_async_copy(
                table_hbm.at[idx_v.at[pl.ds(c * chunk, chunk)]],
                rows_v.at[buf],
                gsem,
            )

        def store_copy(c, buf):
            return pltpu.make_async_copy(
                rows_v.at[buf],
                out_hbm.at[pl.ds(base + c * chunk, chunk)],
                ssem,
            )

        gather_copy(0, 0).start()

        def body(c, carry):
            buf = lax.rem(c, 2)
            nxt = lax.rem(c + 1, 2)

            @pl.when(c > 0)
            def _():
                store_copy(c - 1, nxt).wait()

            @pl.when(c < n_chunks - 1)
            def _():
                gather_copy(c + 1, nxt).start()

            gather_copy(c, buf).wait()
            store_copy(c, buf).start()
            return carry

        lax.fori_loop(0, n_chunks, body, 0)
        store_copy(n_chunks - 1, lax.rem(n_chunks - 1, 2)).wait()

    return gather_kernel


def kernel(x, word_vectors, W):
    projected = _project_table(word_vectors.T, W.T)
    idx = x.T.reshape(N_TOK).astype(jnp.int32)
    out = _make_gather()(projected, idx)
    return out.reshape(TOK_L, TOK_B, EMBED_DIM).transpose(1, 0, 2)


# speedup vs baseline: 1.0904x; 1.0022x over previous
import functools

import jax
import jax.numpy as jnp
from jax import lax
from jax.experimental import pallas as pl
from jax.experimental.pallas import tpu as pltpu
from jax.experimental.pallas import tpu_sc as plsc

VOCAB = 100000
WORD_DIM = 300
EMBED_DIM = 128
TOK_B, TOK_L = 4096, 50
N_TOK = TOK_B * TOK_L

_PROJ_TILE = 12800


def _proj_body(wvt_ref, wt_ref, out_ref):
    out_ref[...] = lax.dot_general(
        wvt_ref[...],
        wt_ref[...],
        dimension_numbers=(((0,), (0,)), ((), ())),
        preferred_element_type=jnp.float32,
    )


def _project_table(wv_t, w_t):
    grid = pl.cdiv(VOCAB, _PROJ_TILE)
    return pl.pallas_call(
        _proj_body,
        grid=(grid,),
        in_specs=[
            pl.BlockSpec((WORD_DIM, _PROJ_TILE), lambda i: (0, i)),
            pl.BlockSpec((WORD_DIM, EMBED_DIM), lambda i: (0, 0)),
        ],
        out_specs=pl.BlockSpec((_PROJ_TILE, EMBED_DIM), lambda i: (i, 0)),
        out_shape=jax.ShapeDtypeStruct((VOCAB, EMBED_DIM), jnp.float32),
    )(wv_t, w_t)


def _make_gather():
    info = plsc.get_sparse_core_info()
    nc, ns = info.num_cores, info.num_subcores
    nw = nc * ns
    b_per_w = N_TOK // nw
    chunk = 400
    n_chunks = b_per_w // chunk
    mesh = plsc.VectorSubcoreMesh(core_axis_name="c", subcore_axis_name="s")

    @functools.partial(
        pl.kernel,
        mesh=mesh,
        out_type=jax.ShapeDtypeStruct((N_TOK, EMBED_DIM), jnp.float32),
        scratch_types=[
            pltpu.VMEM((b_per_w,), jnp.int32),
            pltpu.VMEM((2, chunk, EMBED_DIM), jnp.float32),
            pltpu.SemaphoreType.DMA,
            pltpu.SemaphoreType.DMA,
        ],
    )
    def gather_kernel(table_hbm, idx_hbm, out_hbm, idx_v, rows_v, gsem, ssem):
        wid = lax.axis_index("s") * nc + lax.axis_index("c")
        base = wid * b_per_w
        pltpu.sync_copy(idx_hbm.at[pl.ds(base, b_per_w)], idx_v)

        half = chunk // 2

        def gather_copies(c, buf):
            return [
                pltpu.make_async_copy(
                    table_hbm.at[idx_v.at[pl.ds(c * chunk + h * half, half)]],
                    rows_v.at[buf].at[pl.ds(h * half, half)],
                    gsem,
                )
                for h in range(2)
            ]

        def store_copy(c, buf):
            return pltpu.make_async_copy(
                rows_v.at[buf],
                out_hbm.at[pl.ds(base + c * chunk, chunk)],
                ssem,
            )

        for g in gather_copies(0, 0):
            g.start()

        def body(c, carry):
            buf = lax.rem(c, 2)
            nxt = lax.rem(c + 1, 2)

            @pl.when(c > 0)
            def _():
                store_copy(c - 1, nxt).wait()

            @pl.when(c < n_chunks - 1)
            def _():
                for g in gather_copies(c + 1, nxt):
                    g.start()

            for g in gather_copies(c, buf):
                g.wait()
            store_copy(c, buf).start()
            return carry

        lax.fori_loop(0, n_chunks, body, 0)
        store_copy(n_chunks - 1, lax.rem(n_chunks - 1, 2)).wait()

    return gather_kernel


def kernel(x, word_vectors, W):
    projected = _project_table(word_vectors.T, W.T)
    idx = x.T.reshape(N_TOK).astype(jnp.int32)
    out = _make_gather()(projected, idx)
    return out.reshape(TOK_L, TOK_B, EMBED_DIM).transpose(1, 0, 2)
